# baseline (speedup 1.0000x reference)
import jax
import jax.numpy as jnp
from jax import lax
from jax.experimental import pallas as pl
from jax.experimental.pallas import tpu as pltpu
from jax.experimental.pallas import tpu_sc as plsc

_TOP_K = 2
_TI = 512


def _routing_body(logits_ref, idx_ref, w_ref, lv, iv0, iv1, kv0, kv1, lvec, idxv, wv):
    c = lax.axis_index("c")
    s = lax.axis_index("s")

    @pl.when((c == 0) & (s == 0))
    def _():
        pltpu.sync_copy(logits_ref, lv)
        v = lv[...]
        lanes = lax.iota(jnp.int32, 16)
        neg = jnp.float32(-3e38)
        row0 = jnp.where(lanes < 8, v, neg)
        row1 = jnp.where(lanes >= 8, v, neg)
        sk0, sv0 = plsc.sort_key_val(row0, lanes, descending=True)
        sk1, sv1 = plsc.sort_key_val(row1, lanes, descending=True)
        iv0[...] = sv0
        iv1[...] = sv1
        kv0[...] = sk0
        kv1[...] = sk1
        gidx = jnp.where(lanes < 2, lanes, jnp.maximum(lanes - 2, 0))
        ga = plsc.load_gather(iv0, [gidx])
        gb = plsc.load_gather(iv1, [gidx])
        idx_out = jnp.where(lanes < 2, ga, gb - 8)
        ka = plsc.load_gather(kv0, [gidx])
        kb = plsc.load_gather(kv1, [gidx])
        big_l = jnp.where(lanes < 2, ka, kb)
        lvec[...] = big_l
        lsw = plsc.load_gather(lvec, [jnp.bitwise_xor(lanes, 1)])
        w = 1.0 / (1.0 + jnp.exp(lsw - big_l))
        idxv[...] = idx_out
        wv[...] = w
        pltpu.sync_copy(idxv, idx_ref)
        pltpu.sync_copy(wv, w_ref)


def _sc_route(router_logits):
    B, E = router_logits.shape
    flat = router_logits.reshape(B * E).astype(jnp.float32)
    f = pl.kernel(
        _routing_body,
        out_type=(jax.ShapeDtypeStruct((16,), jnp.int32),
                  jax.ShapeDtypeStruct((16,), jnp.float32)),
        mesh=plsc.VectorSubcoreMesh(core_axis_name="c", subcore_axis_name="s"),
        compiler_params=pltpu.CompilerParams(needs_layout_passes=False),
        scratch_types=[
            pltpu.VMEM((16,), jnp.float32),
            pltpu.VMEM((16,), jnp.int32),
            pltpu.VMEM((16,), jnp.int32),
            pltpu.VMEM((16,), jnp.float32),
            pltpu.VMEM((16,), jnp.float32),
            pltpu.VMEM((16,), jnp.float32),
            pltpu.VMEM((16,), jnp.int32),
            pltpu.VMEM((16,), jnp.float32),
        ],
    )
    idx16, w16 = f(flat)
    return idx16[:2 * _TOP_K].reshape(B, _TOP_K), w16[:2 * _TOP_K].reshape(B, _TOP_K)


def _fused_kernel(idx_ref, vals_ref, x_ref, wg_ref, wu_ref, wd_ref,
                  sg_ref, su_ref, sd_ref, out_ref):
    b = pl.program_id(0)
    k = pl.program_id(1)
    i = pl.program_id(2)

    @pl.when((k == 0) & (i == 0))
    def _init():
        out_ref[0] = jnp.zeros_like(out_ref[0])

    @pl.when(k < _TOP_K)
    def _skill():
        x2 = x_ref[0]
        g = jnp.dot(x2, wg_ref[0], preferred_element_type=jnp.float32)
        u = jnp.dot(x2, wu_ref[0], preferred_element_type=jnp.float32)
        h = jax.nn.gelu(g, approximate=True) * u * vals_ref[b, k]
        out_ref[0] += jnp.dot(h, wd_ref[0], preferred_element_type=jnp.float32)

    @pl.when(k == _TOP_K)
    def _shared():
        x2 = x_ref[0]
        g = jnp.dot(x2, sg_ref[0], preferred_element_type=jnp.float32)
        u = jnp.dot(x2, su_ref[0], preferred_element_type=jnp.float32)
        h = jax.nn.gelu(g, approximate=True) * u
        out_ref[0] += jnp.dot(h, sd_ref[0], preferred_element_type=jnp.float32)


@jax.jit
def kernel(x, router_logits, skill_gate, skill_up, skill_down, shared_gate, shared_up, shared_down):
    B, S, H = x.shape
    E, _, I = skill_gate.shape
    n_i = I // _TI

    idx, vals = _sc_route(router_logits)
    vals = vals.astype(x.dtype)

    def _skill_map(axis):
        def imap(b, k, i, idx, vals):
            kk = jnp.minimum(k, _TOP_K - 1)
            ii = jnp.where(k < _TOP_K, i, n_i - 1)
            e = idx[b, kk]
            return (e, 0, ii) if axis == 0 else (e, ii, 0)
        return imap

    def _shared_map(axis):
        def imap(b, k, i, idx, vals):
            ii = jnp.where(k == _TOP_K, i, 0)
            return (0, 0, ii) if axis == 0 else (0, ii, 0)
        return imap

    out = pl.pallas_call(
        _fused_kernel,
        grid_spec=pltpu.PrefetchScalarGridSpec(
            num_scalar_prefetch=2,
            grid=(B, _TOP_K + 1, n_i),
            in_specs=[
                pl.BlockSpec((1, S, H), lambda b, k, i, idx, vals: (b, 0, 0),
                             pipeline_mode=pl.Buffered(buffer_count=1)),
                pl.BlockSpec((1, H, _TI), _skill_map(0)),
                pl.BlockSpec((1, H, _TI), _skill_map(0)),
                pl.BlockSpec((1, _TI, H), _skill_map(1)),
                pl.BlockSpec((1, H, _TI), _shared_map(0)),
                pl.BlockSpec((1, H, _TI), _shared_map(0)),
                pl.BlockSpec((1, _TI, H), _shared_map(1)),
            ],
            out_specs=pl.BlockSpec((1, S, H), lambda b, k, i, idx, vals: (b, 0, 0),
                                   pipeline_mode=pl.Buffered(buffer_count=1)),
        ),
        out_shape=jax.ShapeDtypeStruct((B, S, H), x.dtype),
    )(idx, vals, x, skill_gate, skill_up, skill_down,
      shared_gate, shared_up, shared_down)

    return out
